# Initial kernel scaffold; baseline (speedup 1.0000x reference)
import functools

import jax
import jax.numpy as jnp
from jax import lax
from jax.experimental import pallas as pl
from jax.experimental.pallas import tpu as pltpu
from jax.experimental.pallas import tpu_sc as plsc

_VOCAB = 100000
_D = 64
_B = 4096
_L = 200
_NC = 2
_NS = 16
_NW = _NC * _NS
_SEQ_PER_W = _B // _NW
_ROWS_PER_W = _SEQ_PER_W * _L
_N = _B * _L
_LANES = 16
_VPR = _D // _LANES


def _sc_embed(tok_flat, table, pe):
    mesh = plsc.VectorSubcoreMesh(core_axis_name="c", subcore_axis_name="s")

    @functools.partial(
        pl.kernel,
        out_type=(
            jax.ShapeDtypeStruct((_N, _D), jnp.float32),
            jax.ShapeDtypeStruct((_N,), jnp.int32),
        ),
        mesh=mesh,
        scratch_types=[
            pltpu.VMEM((_L,), jnp.int32),
            pltpu.VMEM((_L, _D), jnp.float32),
            pltpu.VMEM((_L, _D), jnp.float32),
            pltpu.VMEM((_L,), jnp.float32),
            pltpu.VMEM((_L,), jnp.int32),
            pltpu.SemaphoreType.DMA,
        ],
    )
    def k(tok_hbm, table_hbm, pe_hbm, out_hbm, mask_hbm,
          idx_v, rows_v, pe_v, scale_v, msk_v, sem):
        wid = lax.axis_index("s") * _NC + lax.axis_index("c")
        base = wid * _ROWS_PER_W

        pltpu.sync_copy(pe_hbm.at[pl.ds(0, _L)], pe_v)

        def pe_fix(r, carry):
            for j in range(_VPR):
                sl = pl.ds(j * _LANES, _LANES)
                pe_v[r, sl] = pe_v[r, sl] + 1e-13
            return carry

        lax.fori_loop(0, _L, pe_fix, 0)

        def chunk_body(g, carry):
            cbase = base + g * _L
            pltpu.sync_copy(tok_hbm.at[pl.ds(cbase, _L)], idx_v)
            cp0 = pltpu.async_copy(
                table_hbm.at[idx_v.at[pl.ds(0, 100)]],
                rows_v.at[pl.ds(0, 100)], sem)
            cp1 = pltpu.async_copy(
                table_hbm.at[idx_v.at[pl.ds(100, 100)]],
                rows_v.at[pl.ds(100, 100)], sem)
            for t in range(13):
                o = min(t * _LANES, _L - _LANES)
                sl = pl.ds(o, _LANES)
                nz = idx_v[sl] != 0
                msk_v[sl] = jnp.where(nz, 1, 0).astype(jnp.int32)
                scale_v[sl] = jnp.where(nz, 8.0, 0.0).astype(jnp.float32)
            cp0.wait()
            cp1.wait()

            def row_body(r, carry):
                s = scale_v[r]
                for j in range(_VPR):
                    sl = pl.ds(j * _LANES, _LANES)
                    rows_v[r, sl] = rows_v[r, sl] * s + pe_v[r, sl]
                return carry

            lax.fori_loop(0, _L, row_body, 0)
            pltpu.sync_copy(rows_v, out_hbm.at[pl.ds(cbase, _L)])
            pltpu.sync_copy(msk_v, mask_hbm.at[pl.ds(cbase, _L)])
            return carry

        lax.fori_loop(0, _SEQ_PER_W, chunk_body, 0)

    return k(tok_flat, table, pe)


def kernel(token_tensor, table, pe):
    tok_flat = token_tensor.reshape(-1).astype(jnp.int32)
    out_flat, mask_flat = _sc_embed(tok_flat, table, pe)
    out = out_flat.reshape(_B, _L, _D)
    attention_mask = mask_flat.reshape(_B, _L).astype(jnp.int64)
    return out, attention_mask


# speedup vs baseline: 2.8648x; 2.8648x over previous
import functools

import jax
import jax.numpy as jnp
from jax import lax
from jax.experimental import pallas as pl
from jax.experimental.pallas import tpu as pltpu
from jax.experimental.pallas import tpu_sc as plsc

_VOCAB = 100000
_D = 64
_B = 4096
_L = 200
_NC = 2
_NS = 16
_NW = _NC * _NS
_SEQ_PER_W = _B // _NW
_ROWS_PER_W = _SEQ_PER_W * _L
_N = _B * _L
_LANES = 16
_VPR = _D // _LANES


def _sc_embed(tok_flat, table, pe):
    mesh = plsc.VectorSubcoreMesh(core_axis_name="c", subcore_axis_name="s")

    @functools.partial(
        pl.kernel,
        out_type=(
            jax.ShapeDtypeStruct((_N, _D), jnp.float32),
            jax.ShapeDtypeStruct((_N,), jnp.int32),
        ),
        mesh=mesh,
        compiler_params=pltpu.CompilerParams(use_tc_tiling_on_sc=False),
        scratch_types=[
            pltpu.VMEM((_L,), jnp.int32),
            pltpu.VMEM((_L, _D), jnp.float32),
            pltpu.VMEM((_L, _D), jnp.float32),
            pltpu.VMEM((_L,), jnp.float32),
            pltpu.VMEM((_L,), jnp.int32),
            pltpu.SemaphoreType.DMA,
        ],
    )
    def k(tok_hbm, table_hbm, pe_hbm, out_hbm, mask_hbm,
          idx_v, rows_v, pe_v, scale_v, msk_v, sem):
        wid = lax.axis_index("s") * _NC + lax.axis_index("c")
        base = wid * _ROWS_PER_W

        pltpu.sync_copy(pe_hbm.at[pl.ds(0, _L)], pe_v)

        def pe_fix(r, carry):
            for j in range(_VPR):
                sl = pl.ds(j * _LANES, _LANES)
                pe_v[r, sl] = pe_v[r, sl] + 1e-13
            return carry

        lax.fori_loop(0, _L, pe_fix, 0)

        def chunk_body(g, carry):
            cbase = base + g * _L
            pltpu.sync_copy(tok_hbm.at[pl.ds(cbase, _L)], idx_v)
            cp0 = pltpu.async_copy(
                table_hbm.at[idx_v.at[pl.ds(0, 96)]],
                rows_v.at[pl.ds(0, 96)], sem)
            cp1 = pltpu.async_copy(
                table_hbm.at[idx_v.at[pl.ds(96, 104)]],
                rows_v.at[pl.ds(96, 104)], sem)
            for t in range(13):
                o = min(t * _LANES, _L - _LANES)
                sl = pl.ds(o, _LANES)
                nz = idx_v[sl] != 0
                msk_v[sl] = jnp.where(nz, 1, 0).astype(jnp.int32)
                scale_v[sl] = jnp.where(nz, 8.0, 0.0).astype(jnp.float32)
            cp0.wait()
            cp1.wait()

            def grp_body(t, carry):
                o = t * _LANES
                scale16 = scale_v[pl.ds(o, _LANES)]
                for rr in range(_LANES):
                    s = scale16[rr]
                    for j in range(_VPR):
                        sl = pl.ds(j * _LANES, _LANES)
                        rows_v[o + rr, sl] = (
                            rows_v[o + rr, sl] * s + pe_v[o + rr, sl])
                return carry

            lax.fori_loop(0, _L // _LANES, grp_body, 0)
            tail_scale = scale_v[pl.ds(_L - _LANES, _LANES)]
            for rr in range(_L % _LANES, _LANES):
                s = tail_scale[rr]
                r = _L - _LANES + rr
                for j in range(_VPR):
                    sl = pl.ds(j * _LANES, _LANES)
                    rows_v[r, sl] = rows_v[r, sl] * s + pe_v[r, sl]
            pltpu.sync_copy(rows_v, out_hbm.at[pl.ds(cbase, _L)])
            pltpu.sync_copy(msk_v, mask_hbm.at[pl.ds(cbase, _L)])
            return carry

        lax.fori_loop(0, _SEQ_PER_W, chunk_body, 0)

    return k(tok_flat, table, pe)


def kernel(token_tensor, table, pe):
    tok_flat = token_tensor.reshape(-1).astype(jnp.int32)
    out_flat, mask_flat = _sc_embed(tok_flat, table, pe)
    out = out_flat.reshape(_B, _L, _D)
    attention_mask = mask_flat.reshape(_B, _L).astype(jnp.int64)
    return out, attention_mask
